# baseline (speedup 1.0000x reference)
import functools

import jax
import jax.numpy as jnp
from jax import lax
from jax.experimental import pallas as pl
from jax.experimental.pallas import tpu as pltpu
from jax.experimental.pallas import tpu_sc as plsc

EMBED = 64
B = 4096
L = 50
NW = 32
BW = B // NW

_mesh = plsc.VectorSubcoreMesh(core_axis_name="c", subcore_axis_name="s")


@functools.partial(
    pl.kernel,
    mesh=_mesh,
    out_type=jax.ShapeDtypeStruct((2, L, EMBED // 8, NW, 8, BW), jnp.float32),
    compiler_params=pltpu.CompilerParams(use_tc_tiling_on_sc=False,
                                         needs_layout_passes=False),
    scratch_types=(
        [pltpu.VMEM((BW, L), jnp.int32)]
        + [pltpu.VMEM((BW,), jnp.int32) for _ in range(2)]
        + [pltpu.VMEM((BW, EMBED), jnp.float32) for _ in range(2)]
        + [pltpu.VMEM((EMBED // 8, 8, BW), jnp.float32) for _ in range(2)]
        + [pltpu.SemaphoreType.DMA for _ in range(4)]
    ),
)
def _emb_lookup(src_tok, tgt_tok, src_tab, tgt_tab, out, tok2d, *scr):
    idxb = scr[0:2]
    rows = scr[2:4]
    tbuf = scr[4:6]
    gsem = scr[6:8]
    ssem = scr[8:10]
    wid = lax.axis_index("s") * 2 + lax.axis_index("c")
    row0 = wid * BW

    toks = (src_tok, tgt_tok)
    tabs = (src_tab, tgt_tab)
    ii = lax.broadcasted_iota(jnp.int32, (16,), 0)
    bvecs = [b0 * 16 + ii for b0 in range(8)]

    def build_idx(l, s):
        lvec = jnp.broadcast_to(l, (16,)).astype(jnp.int32)
        for b0 in range(8):
            idxb[s][pl.ds(b0 * 16, 16)] = plsc.load_gather(
                tok2d, [bvecs[b0], lvec])

    def transpose(s):
        def ebody(e_hi, carry):
            for e_lo in range(8):
                evec = jnp.broadcast_to(e_hi * 8 + e_lo, (16,)).astype(
                    jnp.int32)
                for b0 in range(8):
                    tbuf[s][e_hi, e_lo, pl.ds(b0 * 16, 16)] = (
                        plsc.load_gather(rows[s], [bvecs[b0], evec]))
            return carry
        lax.fori_loop(0, EMBED // 8, ebody, 0)

    for t in range(2):
        pltpu.sync_copy(toks[t].at[pl.ds(row0, BW)], tok2d)
        for s in range(2):
            build_idx(s, s)
            pltpu.async_copy(tabs[t].at[idxb[s]], rows[s], gsem[s])

        def body(g, carry):
            for s in range(2):
                k = 2 * g + s
                pltpu.make_async_copy(
                    tabs[t].at[pl.ds(0, BW)], rows[s], gsem[s]).wait()
                @pl.when(g >= 1)
                def _():
                    pltpu.make_async_copy(
                        out.at[t, 0, :, 0], tbuf[s], ssem[s]).wait()
                transpose(s)
                pltpu.async_copy(tbuf[s], out.at[t, k, :, wid], ssem[s])

                @pl.when(k + 2 < L)
                def _():
                    build_idx(k + 2, s)
                    pltpu.async_copy(tabs[t].at[idxb[s]], rows[s], gsem[s])
            return carry

        lax.fori_loop(0, L // 2, body, 0)
        for s in range(2):
            pltpu.make_async_copy(out.at[t, 0, :, 0], tbuf[s], ssem[s]).wait()


def kernel(src_tokens, tgt_tokens, src_table, tgt_table):
    o6 = _emb_lookup(src_tokens.astype(jnp.int32),
                     tgt_tokens.astype(jnp.int32), src_table, tgt_table)
    return jnp.transpose(o6, (0, 3, 5, 1, 2, 4)).reshape(2, B, L, EMBED)


# speedup vs baseline: 1.3478x; 1.3478x over previous
import functools

import jax
import jax.numpy as jnp
from jax import lax
from jax.experimental import pallas as pl
from jax.experimental.pallas import tpu as pltpu
from jax.experimental.pallas import tpu_sc as plsc

EMBED = 64
B = 4096
L = 50
NW = 32
BW = B // NW
EH = EMBED // 8

_mesh = plsc.VectorSubcoreMesh(core_axis_name="c", subcore_axis_name="s")


@functools.partial(
    pl.kernel,
    mesh=_mesh,
    out_type=jax.ShapeDtypeStruct((2, L, EH * NW, 8 * BW), jnp.float32),
    compiler_params=pltpu.CompilerParams(use_tc_tiling_on_sc=False,
                                         needs_layout_passes=False),
    scratch_types=(
        [pltpu.VMEM((BW, L), jnp.int32)]
        + [pltpu.VMEM((BW,), jnp.int32) for _ in range(2)]
        + [pltpu.VMEM((BW, EMBED), jnp.float32) for _ in range(2)]
        + [pltpu.VMEM((EH, 8 * BW), jnp.float32) for _ in range(2)]
        + [pltpu.SemaphoreType.DMA for _ in range(4)]
    ),
)
def _emb_lookup(src_tok, tgt_tok, src_tab, tgt_tab, out, tok2d, *scr):
    idxb = scr[0:2]
    rows = scr[2:4]
    tbuf = scr[4:6]
    gsem = scr[6:8]
    ssem = scr[8:10]
    wid = lax.axis_index("s") * 2 + lax.axis_index("c")
    row0 = wid * BW

    toks = (src_tok, tgt_tok)
    tabs = (src_tab, tgt_tab)
    ii = lax.broadcasted_iota(jnp.int32, (16,), 0)
    bvecs = [b0 * 16 + ii for b0 in range(8)]

    def build_idx(l, s):
        lvec = jnp.broadcast_to(l, (16,)).astype(jnp.int32)
        for b0 in range(8):
            idxb[s][pl.ds(b0 * 16, 16)] = plsc.load_gather(
                tok2d, [bvecs[b0], lvec])

    def transpose(s):
        @plsc.parallel_loop(0, EH, unroll=2)
        def _(e_hi):
            for e_lo in range(8):
                evec = jnp.broadcast_to(e_hi * 8 + e_lo, (16,)).astype(
                    jnp.int32)
                for b0 in range(8):
                    tbuf[s][e_hi, pl.ds(e_lo * BW + b0 * 16, 16)] = (
                        plsc.load_gather(rows[s], [bvecs[b0], evec]))

    for t in range(2):
        pltpu.sync_copy(toks[t].at[pl.ds(row0, BW)], tok2d)
        for s in range(2):
            build_idx(s, s)
            pltpu.async_copy(tabs[t].at[idxb[s]], rows[s], gsem[s])

        def body(g, carry):
            for s in range(2):
                k = 2 * g + s
                pltpu.make_async_copy(
                    tabs[t].at[pl.ds(0, BW)], rows[s], gsem[s]).wait()
                @pl.when(g >= 1)
                def _():
                    pltpu.make_async_copy(
                        out.at[t, 0].at[pl.ds(0, EH)], tbuf[s],
                        ssem[s]).wait()
                transpose(s)
                for e_hi in range(EH):
                    pltpu.async_copy(tbuf[s].at[e_hi],
                                     out.at[t, k, e_hi * NW + wid], ssem[s])

                @pl.when(k + 2 < L)
                def _():
                    build_idx(k + 2, s)
                    pltpu.async_copy(tabs[t].at[idxb[s]], rows[s], gsem[s])
            return carry

        lax.fori_loop(0, L // 2, body, 0)
        for s in range(2):
            pltpu.make_async_copy(out.at[t, 0].at[pl.ds(0, EH)], tbuf[s],
                                  ssem[s]).wait()


def kernel(src_tokens, tgt_tokens, src_table, tgt_table):
    o = _emb_lookup(src_tokens.astype(jnp.int32),
                    tgt_tokens.astype(jnp.int32), src_table, tgt_table)
    o6 = o.reshape(2, L, EH, NW, 8, BW)
    return jnp.transpose(o6, (0, 3, 5, 1, 2, 4)).reshape(2, B, L, EMBED)


# speedup vs baseline: 3.2216x; 2.3903x over previous
import functools

import jax
import jax.numpy as jnp
from jax import lax
from jax.experimental import pallas as pl
from jax.experimental.pallas import tpu as pltpu
from jax.experimental.pallas import tpu_sc as plsc

EMBED = 64
B = 4096
L = 50
NW = 32
BW = B // NW
EH = EMBED // 8
TP = BW + 9

_mesh = plsc.VectorSubcoreMesh(core_axis_name="c", subcore_axis_name="s")


@functools.partial(
    pl.kernel,
    mesh=_mesh,
    out_type=jax.ShapeDtypeStruct((2, L, EH, NW, 8, BW), jnp.float32),
    compiler_params=pltpu.CompilerParams(use_tc_tiling_on_sc=False,
                                         needs_layout_passes=False),
    scratch_types=(
        [pltpu.VMEM((BW, L), jnp.int32)]
        + [pltpu.VMEM((BW,), jnp.int32) for _ in range(2)]
        + [pltpu.VMEM((BW, EMBED), jnp.float32) for _ in range(2)]
        + [pltpu.VMEM((EMBED, TP), jnp.float32) for _ in range(2)]
        + [pltpu.SemaphoreType.DMA for _ in range(4)]
    ),
)
def _emb_lookup(src_tok, tgt_tok, src_tab, tgt_tab, out, tok2d, *scr):
    idxb = scr[0:2]
    rows = scr[2:4]
    tbuf = scr[4:6]
    gsem = scr[6:8]
    ssem = scr[8:10]
    wid = lax.axis_index("s") * 2 + lax.axis_index("c")
    row0 = wid * BW

    toks = (src_tok, tgt_tok)
    tabs = (src_tab, tgt_tab)
    ii = lax.broadcasted_iota(jnp.int32, (16,), 0)
    bvecs = [b0 * 16 + ii for b0 in range(8)]
    evecs = [e0 * 16 + ii for e0 in range(EMBED // 16)]

    def build_idx(l, s):
        lvec = jnp.broadcast_to(l, (16,)).astype(jnp.int32)
        for b0 in range(8):
            idxb[s][pl.ds(b0 * 16, 16)] = plsc.load_gather(
                tok2d, [bvecs[b0], lvec])

    def transpose(s):
        @plsc.parallel_loop(0, BW, unroll=4)
        def _(b):
            bvec = jnp.broadcast_to(b, (16,)).astype(jnp.int32)
            for e0 in range(EMBED // 16):
                plsc.store_scatter(tbuf[s], [evecs[e0], bvec],
                                   rows[s][b, pl.ds(e0 * 16, 16)])

    def drain_stores(t, s):
        for _ in range(EH):
            pltpu.make_async_copy(out.at[t, 0, 0, 0],
                                  tbuf[s].at[pl.ds(0, 8), pl.ds(0, BW)],
                                  ssem[s]).wait()

    for t in range(2):
        pltpu.sync_copy(toks[t].at[pl.ds(row0, BW)], tok2d)
        for s in range(2):
            build_idx(s, s)
            pltpu.async_copy(tabs[t].at[idxb[s]], rows[s], gsem[s])

        def body(g, carry):
            for s in range(2):
                k = 2 * g + s
                pltpu.make_async_copy(
                    tabs[t].at[pl.ds(0, BW)], rows[s], gsem[s]).wait()
                @pl.when(g >= 1)
                def _():
                    drain_stores(t, s)
                transpose(s)
                for e_hi in range(EH):
                    pltpu.async_copy(
                        tbuf[s].at[pl.ds(e_hi * 8, 8), pl.ds(0, BW)],
                        out.at[t, k, e_hi, wid], ssem[s])

                @pl.when(k + 2 < L)
                def _():
                    build_idx(k + 2, s)
                    pltpu.async_copy(tabs[t].at[idxb[s]], rows[s], gsem[s])
            return carry

        lax.fori_loop(0, L // 2, body, 0)
        for s in range(2):
            drain_stores(t, s)


def kernel(src_tokens, tgt_tokens, src_table, tgt_table):
    o6 = _emb_lookup(src_tokens.astype(jnp.int32),
                     tgt_tokens.astype(jnp.int32), src_table, tgt_table)
    return jnp.transpose(o6, (0, 3, 5, 1, 2, 4)).reshape(2, B, L, EMBED)


# speedup vs baseline: 3.2296x; 1.0025x over previous
import functools

import jax
import jax.numpy as jnp
from jax import lax
from jax.experimental import pallas as pl
from jax.experimental.pallas import tpu as pltpu
from jax.experimental.pallas import tpu_sc as plsc

EMBED = 64
B = 4096
L = 50
NW = 32
BW = B // NW
EH = EMBED // 8
TP = BW + 9

_mesh = plsc.VectorSubcoreMesh(core_axis_name="c", subcore_axis_name="s")


@functools.partial(
    pl.kernel,
    mesh=_mesh,
    out_type=jax.ShapeDtypeStruct((2, L, EH, NW, 8, BW), jnp.float32),
    compiler_params=pltpu.CompilerParams(use_tc_tiling_on_sc=False,
                                         needs_layout_passes=False),
    scratch_types=(
        [pltpu.VMEM((BW, L), jnp.int32)]
        + [pltpu.VMEM((BW,), jnp.int32) for _ in range(2)]
        + [pltpu.VMEM((BW, EMBED), jnp.float32) for _ in range(2)]
        + [pltpu.VMEM((EMBED, TP), jnp.float32) for _ in range(2)]
        + [pltpu.SemaphoreType.DMA for _ in range(4)]
    ),
)
def _emb_lookup(src_tok, tgt_tok, src_tab, tgt_tab, out, tok2d, *scr):
    idxb = scr[0:2]
    rows = scr[2:4]
    tbuf = scr[4:6]
    gsem = scr[6:8]
    ssem = scr[8:10]
    wid = lax.axis_index("s") * 2 + lax.axis_index("c")
    row0 = wid * BW

    toks = (src_tok, tgt_tok)
    tabs = (src_tab, tgt_tab)
    ii = lax.broadcasted_iota(jnp.int32, (16,), 0)
    bvecs = [b0 * 16 + ii for b0 in range(8)]
    evecs = [e0 * 16 + ii for e0 in range(EMBED // 16)]

    def build_idx(l, s):
        lvec = jnp.broadcast_to(l, (16,)).astype(jnp.int32)
        for b0 in range(8):
            idxb[s][pl.ds(b0 * 16, 16)] = plsc.load_gather(
                tok2d, [bvecs[b0], lvec])

    def transpose(s):
        @plsc.parallel_loop(0, BW, unroll=8)
        def _(b):
            bvec = jnp.broadcast_to(b, (16,)).astype(jnp.int32)
            for e0 in range(EMBED // 16):
                plsc.store_scatter(tbuf[s], [evecs[e0], bvec],
                                   rows[s][b, pl.ds(e0 * 16, 16)])

    def drain_stores(t, s):
        for _ in range(EH):
            pltpu.make_async_copy(out.at[t, 0, 0, 0],
                                  tbuf[s].at[pl.ds(0, 8), pl.ds(0, BW)],
                                  ssem[s]).wait()

    for t in range(2):
        pltpu.sync_copy(toks[t].at[pl.ds(row0, BW)], tok2d)
        for s in range(2):
            build_idx(s, s)
            pltpu.async_copy(tabs[t].at[idxb[s]], rows[s], gsem[s])

        def body(g, carry):
            for s in range(2):
                k = 2 * g + s
                pltpu.make_async_copy(
                    tabs[t].at[pl.ds(0, BW)], rows[s], gsem[s]).wait()
                @pl.when(g >= 1)
                def _():
                    drain_stores(t, s)
                transpose(s)
                for e_hi in range(EH):
                    pltpu.async_copy(
                        tbuf[s].at[pl.ds(e_hi * 8, 8), pl.ds(0, BW)],
                        out.at[t, k, e_hi, wid], ssem[s])

                @pl.when(k + 2 < L)
                def _():
                    build_idx(k + 2, s)
                    pltpu.async_copy(tabs[t].at[idxb[s]], rows[s], gsem[s])
            return carry

        lax.fori_loop(0, L // 2, body, 0)
        for s in range(2):
            drain_stores(t, s)


def kernel(src_tokens, tgt_tokens, src_table, tgt_table):
    o6 = _emb_lookup(src_tokens.astype(jnp.int32),
                     tgt_tokens.astype(jnp.int32), src_table, tgt_table)
    return jnp.transpose(o6, (0, 3, 5, 1, 2, 4)).reshape(2, B, L, EMBED)


# speedup vs baseline: 3.2303x; 1.0002x over previous
import functools

import jax
import jax.numpy as jnp
from jax import lax
from jax.experimental import pallas as pl
from jax.experimental.pallas import tpu as pltpu
from jax.experimental.pallas import tpu_sc as plsc

EMBED = 64
B = 4096
L = 50
NW = 32
BW = B // NW
EH = EMBED // 8
TP = BW + 9

_mesh = plsc.VectorSubcoreMesh(core_axis_name="c", subcore_axis_name="s")


@functools.partial(
    pl.kernel,
    mesh=_mesh,
    out_type=jax.ShapeDtypeStruct((2, L, EH, NW, 8, BW), jnp.float32),
    compiler_params=pltpu.CompilerParams(use_tc_tiling_on_sc=False,
                                         needs_layout_passes=False),
    scratch_types=(
        [pltpu.VMEM((BW, L), jnp.int32)]
        + [pltpu.VMEM((BW,), jnp.int32) for _ in range(2)]
        + [pltpu.VMEM((BW, EMBED), jnp.float32) for _ in range(2)]
        + [pltpu.VMEM((EMBED, TP), jnp.float32) for _ in range(2)]
        + [pltpu.SemaphoreType.DMA for _ in range(4)]
    ),
)
def _emb_lookup(src_tok, tgt_tok, src_tab, tgt_tab, out, tok2d, *scr):
    idxb = scr[0:2]
    rows = scr[2:4]
    tbuf = scr[4:6]
    gsem = scr[6:8]
    ssem = scr[8:10]
    wid = lax.axis_index("s") * 2 + lax.axis_index("c")
    row0 = wid * BW

    toks = (src_tok, tgt_tok)
    tabs = (src_tab, tgt_tab)
    ii = lax.broadcasted_iota(jnp.int32, (16,), 0)
    bvecs = [b0 * 16 + ii for b0 in range(8)]
    evecs = [e0 * 16 + ii for e0 in range(EMBED // 16)]

    def build_idx(l, s):
        lvec = jnp.broadcast_to(l, (16,)).astype(jnp.int32)
        for b0 in range(8):
            idxb[s][pl.ds(b0 * 16, 16)] = plsc.load_gather(
                tok2d, [bvecs[b0], lvec])

    def transpose(s):
        @plsc.parallel_loop(0, BW, unroll=4)
        def _(b):
            bvec = jnp.broadcast_to(b, (16,)).astype(jnp.int32)
            for e0 in range(EMBED // 16):
                plsc.store_scatter(tbuf[s], [evecs[e0], bvec],
                                   rows[s][b, pl.ds(e0 * 16, 16)])

    def drain_stores(t, s):
        for _ in range(EH):
            pltpu.make_async_copy(out.at[t, 0, 0, 0],
                                  tbuf[s].at[pl.ds(0, 8), pl.ds(0, BW)],
                                  ssem[s]).wait()

    for t in range(2):
        pltpu.sync_copy(toks[t].at[pl.ds(row0, BW)], tok2d)
        for s in range(2):
            build_idx(s, s)
            pltpu.async_copy(tabs[t].at[idxb[s]], rows[s], gsem[s])

        def body(g, carry):
            for s in range(2):
                k = 2 * g + s
                pltpu.make_async_copy(
                    tabs[t].at[pl.ds(0, BW)], rows[s], gsem[s]).wait()
                @pl.when(g >= 1)
                def _():
                    drain_stores(t, s)
                transpose(s)
                for e_hi in range(EH):
                    pltpu.async_copy(
                        tbuf[s].at[pl.ds(e_hi * 8, 8), pl.ds(0, BW)],
                        out.at[t, k, e_hi, wid], ssem[s])

                @pl.when(k + 2 < L)
                def _():
                    build_idx(k + 2, s)
                    pltpu.async_copy(tabs[t].at[idxb[s]], rows[s], gsem[s])
            return carry

        lax.fori_loop(0, L // 2, body, 0)
        for s in range(2):
            drain_stores(t, s)


def kernel(src_tokens, tgt_tokens, src_table, tgt_table):
    o6 = _emb_lookup(src_tokens.astype(jnp.int32),
                     tgt_tokens.astype(jnp.int32), src_table, tgt_table)
    return jnp.transpose(o6, (0, 3, 5, 1, 2, 4)).reshape(2, B, L, EMBED)


# speedup vs baseline: 3.4010x; 1.0529x over previous
import functools

import jax
import jax.numpy as jnp
from jax import lax
from jax.experimental import pallas as pl
from jax.experimental.pallas import tpu as pltpu
from jax.experimental.pallas import tpu_sc as plsc

EMBED = 64
B = 4096
L = 50
NW = 32
BW = B // NW
EH = EMBED // 8
TP = BW + 9
NCH = L // 2

_mesh = plsc.VectorSubcoreMesh(core_axis_name="c", subcore_axis_name="s")


@functools.partial(
    pl.kernel,
    mesh=_mesh,
    out_type=jax.ShapeDtypeStruct((2, L, EH, NW, 8, BW), jnp.float32),
    compiler_params=pltpu.CompilerParams(use_tc_tiling_on_sc=False,
                                         needs_layout_passes=False),
    scratch_types=(
        [pltpu.VMEM((BW, L), jnp.int32)]
        + [pltpu.VMEM((2 * BW,), jnp.int32) for _ in range(2)]
        + [pltpu.VMEM((2 * BW, EMBED), jnp.float32) for _ in range(2)]
        + [pltpu.VMEM((EMBED, TP), jnp.float32) for _ in range(4)]
        + [pltpu.SemaphoreType.DMA for _ in range(4)]
    ),
)
def _emb_lookup(src_tok, tgt_tok, src_tab, tgt_tab, out, tok2d, *scr):
    idxb = scr[0:2]
    rows = scr[2:4]
    tbufs = scr[4:8]
    gsem = scr[8:10]
    ssem = scr[10:12]
    wid = lax.axis_index("s") * 2 + lax.axis_index("c")
    row0 = wid * BW

    toks = (src_tok, tgt_tok)
    tabs = (src_tab, tgt_tab)
    ii = lax.broadcasted_iota(jnp.int32, (16,), 0)
    bvecs = [b0 * 16 + ii for b0 in range(8)]
    evecs = [e0 * 16 + ii for e0 in range(EMBED // 16)]

    def build_idx(k, s):
        for h in range(2):
            lvec = jnp.broadcast_to(2 * k + h, (16,)).astype(jnp.int32)
            for b0 in range(8):
                idxb[s][pl.ds(h * BW + b0 * 16, 16)] = plsc.load_gather(
                    tok2d, [bvecs[b0], lvec])

    def transpose(s):
        for h in range(2):
            tb = tbufs[s * 2 + h]

            @plsc.parallel_loop(0, BW, unroll=4)
            def _(b):
                bvec = jnp.broadcast_to(b, (16,)).astype(jnp.int32)
                for e0 in range(EMBED // 16):
                    plsc.store_scatter(
                        tb, [evecs[e0], bvec],
                        rows[s][h * BW + b, pl.ds(e0 * 16, 16)])

    def start_stores(t, k, s):
        for h in range(2):
            for e_hi in range(EH):
                pltpu.async_copy(
                    tbufs[s * 2 + h].at[pl.ds(e_hi * 8, 8), pl.ds(0, BW)],
                    out.at[t, 2 * k + h, e_hi, wid], ssem[s])

    def drain_stores(t, s):
        for _ in range(2 * EH):
            pltpu.make_async_copy(out.at[t, 0, 0, 0],
                                  tbufs[s * 2].at[pl.ds(0, 8), pl.ds(0, BW)],
                                  ssem[s]).wait()

    def wait_gather(t, s):
        pltpu.make_async_copy(tabs[t].at[pl.ds(0, 2 * BW)], rows[s],
                              gsem[s]).wait()

    for t in range(2):
        pltpu.sync_copy(toks[t].at[pl.ds(row0, BW)], tok2d)
        for s in range(2):
            build_idx(s, s)
            pltpu.async_copy(tabs[t].at[idxb[s]], rows[s], gsem[s])

        def body(g, carry):
            for s in range(2):
                k = 2 * g + s
                wait_gather(t, s)

                @pl.when(g >= 1)
                def _():
                    drain_stores(t, s)

                transpose(s)
                start_stores(t, k, s)

                @pl.when(k + 2 < NCH)
                def _():
                    build_idx(k + 2, s)
                    pltpu.async_copy(tabs[t].at[idxb[s]], rows[s], gsem[s])
            return carry

        lax.fori_loop(0, NCH // 2, body, 0)
        wait_gather(t, 0)
        drain_stores(t, 0)
        transpose(0)
        start_stores(t, NCH - 1, 0)
        for s in range(2):
            drain_stores(t, s)


def kernel(src_tokens, tgt_tokens, src_table, tgt_table):
    o6 = _emb_lookup(src_tokens.astype(jnp.int32),
                     tgt_tokens.astype(jnp.int32), src_table, tgt_table)
    return jnp.transpose(o6, (0, 3, 5, 1, 2, 4)).reshape(2, B, L, EMBED)
